# Initial kernel scaffold; baseline (speedup 1.0000x reference)
import jax
import jax.numpy as jnp
from jax.experimental import pallas as pl


def kernel(x, edge_index, edge_attr, ln_g, ln_b, q_W1, q_b1, q_W2, q_b2, k_W1, k_b1, k_W2, k_b2, v_W1, v_b1, v_W2, v_b2, sp_W1, sp_b1, sp_W2, sp_b2, rc_W, rc_b, ff_W1, ff_b1, ff_W2, ff_b2):
    raise NotImplementedError("write your pallas kernel here")


# speedup vs baseline: 2.5672x; 2.5672x over previous
import functools

import jax
import jax.numpy as jnp
from jax import lax
from jax.experimental import pallas as pl
from jax.experimental.pallas import tpu as pltpu
from jax.experimental.pallas import tpu_sc as plsc

HID = 128
NH = 8
DPH = 16
N_NODES = 10000
N_EDGES = 160000

_NC = 2
_NS = 16
_NW = _NC * _NS
_CHUNK = 128
_NCHUNKS = N_EDGES // _CHUNK
_ITERS = -(-_NCHUNKS // _NW)
_NPS = N_NODES // _NC
_TBL = 5120
_ZCH = _TBL // _CHUNK

_EB = 2000
_NB = 2000
_FB = 1000


def _erf(z):
    s = jnp.sign(z)
    a = jnp.abs(z)
    t = 1.0 / (1.0 + 0.3275911 * a)
    poly = t * (0.254829592 + t * (-0.284496736 + t * (1.421413741
           + t * (-1.453152027 + t * 1.061405429))))
    return s * (1.0 - poly * jnp.exp(-a * a))


def _gelu(h):
    return 0.5 * h * (1.0 + _erf(h * 0.7071067811865475))


def _ln_v(xv, g, b):
    mu = jnp.mean(xv, axis=1, keepdims=True)
    var = jnp.mean((xv - mu) ** 2, axis=1, keepdims=True)
    return (xv - mu) * lax.rsqrt(var + 1e-5) * g + b


def _ln_body(x_ref, g_ref, b_ref, o_ref):
    o_ref[...] = _ln_v(x_ref[...], g_ref[...], b_ref[...])


def _ln_call(x, g, b):
    grid = (N_NODES // _NB,)
    return pl.pallas_call(
        _ln_body,
        grid=grid,
        in_specs=[
            pl.BlockSpec((_NB, HID), lambda i: (i, 0)),
            pl.BlockSpec((1, HID), lambda i: (0, 0)),
            pl.BlockSpec((1, HID), lambda i: (0, 0)),
        ],
        out_specs=pl.BlockSpec((_NB, HID), lambda i: (i, 0)),
        out_shape=jax.ShapeDtypeStruct((N_NODES, HID), jnp.float32),
    )(x, g, b)


def _gather_body(xn_hbm, src_hbm, dst_hbm, srcf_out, dstf_out,
                 idx_s, idx_d, rows_s, rows_d, sem_s, sem_d):
    c = lax.axis_index("c")
    s = lax.axis_index("s")
    wid = s * _NC + c

    def body(j, carry):
        chunk = wid + j * _NW

        @pl.when(chunk < _NCHUNKS)
        def _():
            base = chunk * _CHUNK
            pltpu.sync_copy(src_hbm.at[pl.ds(base, _CHUNK)], idx_s)
            pltpu.sync_copy(dst_hbm.at[pl.ds(base, _CHUNK)], idx_d)
            cs = pltpu.async_copy(xn_hbm.at[idx_s], rows_s, sem_s)
            cd = pltpu.async_copy(xn_hbm.at[idx_d], rows_d, sem_d)
            cs.wait()
            cd.wait()
            pltpu.sync_copy(rows_s, srcf_out.at[pl.ds(base, _CHUNK)])
            pltpu.sync_copy(rows_d, dstf_out.at[pl.ds(base, _CHUNK)])
        return carry

    lax.fori_loop(0, _ITERS, body, 0)


def _gather_call(xn, src, dst):
    mesh = plsc.VectorSubcoreMesh(core_axis_name="c", subcore_axis_name="s", num_cores=_NC, num_subcores=_NS)
    f = pl.kernel(
        _gather_body,
        out_type=[
            jax.ShapeDtypeStruct((N_EDGES, HID), jnp.float32),
            jax.ShapeDtypeStruct((N_EDGES, HID), jnp.float32),
        ],
        mesh=mesh,
        scratch_types=[
            pltpu.VMEM((_CHUNK,), jnp.int32),
            pltpu.VMEM((_CHUNK,), jnp.int32),
            pltpu.VMEM((_CHUNK, HID), jnp.float32),
            pltpu.VMEM((_CHUNK, HID), jnp.float32),
            pltpu.SemaphoreType.DMA,
            pltpu.SemaphoreType.DMA,
        ],
    )
    return f(xn, src, dst)


def _edge_body(srcf_ref, dstf_ref, ea_ref,
               W1s_ref, W1d_ref, W1m_ref, b1_ref,
               qW2_ref, qb2_ref, kW2_ref, kb2_ref, vW2_ref, vb2_ref,
               spW1_ref, spb1_ref, spW2_ref, spb2_ref,
               R_ref, D_ref, Eexp_ref, out_ref):
    f32 = jnp.float32
    ea = ea_ref[...]
    h = jnp.dot(srcf_ref[...], W1s_ref[...], preferred_element_type=f32)
    h = h + jnp.dot(dstf_ref[...], W1d_ref[...], preferred_element_type=f32)
    h = h + jnp.dot(ea, W1m_ref[...], preferred_element_type=f32)
    h = h + b1_ref[...]
    h = _gelu(h)

    Q = jnp.dot(h[:, 0:256], qW2_ref[...], preferred_element_type=f32) + qb2_ref[...]
    K = jnp.dot(h[:, 256:512], kW2_ref[...], preferred_element_type=f32) + kb2_ref[...]
    V = jnp.dot(h[:, 512:768], vW2_ref[...], preferred_element_type=f32) + vb2_ref[...]

    sp_h = jnp.maximum(jnp.dot(ea, spW1_ref[...], preferred_element_type=f32)
                       + spb1_ref[...], 0.0)
    bias8 = jnp.dot(sp_h, spW2_ref[...], preferred_element_type=f32) + spb2_ref[...]

    R = R_ref[...]
    sg = []
    for g in range(NH):
        Kg = K[:, g * DPH:(g + 1) * DPH]
        Kt = jnp.concatenate([Kg] * NH, axis=1)
        sg.append(jnp.dot(Q * Kt, R, preferred_element_type=f32))
    S64 = jnp.concatenate(sg, axis=1)
    bias64 = jnp.concatenate([bias8] * NH, axis=1)
    S = S64 * 0.25 + bias64

    m = jnp.max(S, axis=1, keepdims=True)
    Ee = jnp.exp(S - m)
    denom = jnp.dot(Ee, D_ref[...], preferred_element_type=f32)
    dinv = 1.0 / denom

    Eexp = Eexp_ref[...]
    msgs = jnp.zeros(out_ref.shape, f32)
    for g in range(NH):
        Ag = Ee[:, g * NH:(g + 1) * NH]
        Wg = jnp.dot(Ag, Eexp, preferred_element_type=f32)
        Vg = V[:, g * DPH:(g + 1) * DPH]
        Vt = jnp.concatenate([Vg] * NH, axis=1)
        msgs = msgs + Wg * Vt * dinv[:, g:g + 1]
    out_ref[...] = msgs


def _edge_call(srcf, dstf, ea, weights):
    (W1s, W1d, W1m, b1, qW2, qb2, kW2, kb2, vW2, vb2,
     spW1, spb1, spW2, spb2, R, D, Eexp) = weights
    grid = (N_EDGES // _EB,)
    zero2 = lambda i: (0, 0)
    full = lambda a: pl.BlockSpec(a.shape, zero2)
    return pl.pallas_call(
        _edge_body,
        grid=grid,
        in_specs=[
            pl.BlockSpec((_EB, HID), lambda i: (i, 0)),
            pl.BlockSpec((_EB, HID), lambda i: (i, 0)),
            pl.BlockSpec((_EB, 16), lambda i: (i, 0)),
            full(W1s), full(W1d), full(W1m), full(b1),
            full(qW2), full(qb2), full(kW2), full(kb2), full(vW2), full(vb2),
            full(spW1), full(spb1), full(spW2), full(spb2),
            full(R), full(D), full(Eexp),
        ],
        out_specs=pl.BlockSpec((_EB, HID), lambda i: (i, 0)),
        out_shape=jax.ShapeDtypeStruct((N_EDGES, HID), jnp.float32),
        compiler_params=pltpu.CompilerParams(
            dimension_semantics=("arbitrary",)),
    )(srcf, dstf, ea, W1s, W1d, W1m, b1, qW2, qb2, kW2, kb2, vW2, vb2,
      spW1, spb1, spW2, spb2, R, D, Eexp)


def _scatter_body(msgs_hbm, dst_hbm, zeros_hbm, ones_hbm,
                  agg_out, cnt_out,
                  idx_v, idxl_v, rows_v, ones_v, acc_sh, cnt_sh):
    c = lax.axis_index("c")
    s = lax.axis_index("s")
    base_node = c * _NPS

    pltpu.sync_copy(zeros_hbm, rows_v)
    pltpu.sync_copy(ones_hbm, ones_v)

    def zbody(j, carry):
        chunk = s + j * _NS

        @pl.when(chunk < _ZCH)
        def _():
            r = chunk * _CHUNK
            pltpu.sync_copy(rows_v, acc_sh.at[pl.ds(r, _CHUNK)])
            pltpu.sync_copy(rows_v, cnt_sh.at[pl.ds(r, _CHUNK)])
        return carry

    lax.fori_loop(0, -(-_ZCH // _NS), zbody, 0)
    plsc.subcore_barrier()

    def body(j, carry):
        chunk = s + j * _NS

        @pl.when(chunk < _NCHUNKS)
        def _():
            base = chunk * _CHUNK
            pltpu.sync_copy(dst_hbm.at[pl.ds(base, _CHUNK)], idx_v)
            pltpu.sync_copy(msgs_hbm.at[pl.ds(base, _CHUNK)], rows_v)
            for g in range(_CHUNK // 16):
                v = idx_v[pl.ds(g * 16, 16)]
                loc = v - base_node
                ok = (loc >= 0) & (loc < _NPS)
                idxl_v[pl.ds(g * 16, 16)] = jnp.where(ok, loc, _NPS)
            pltpu.sync_copy(rows_v, acc_sh.at[idxl_v], add=True)
            pltpu.sync_copy(ones_v, cnt_sh.at[idxl_v], add=True)
        return carry

    lax.fori_loop(0, -(-_NCHUNKS // _NS), body, 0)
    plsc.subcore_barrier()

    def dbody(j, carry):
        chunk = s + j * _NS

        @pl.when(chunk < _ZCH)
        def _():
            r = chunk * _CHUNK
            pltpu.sync_copy(acc_sh.at[pl.ds(r, _CHUNK)], rows_v)
            pltpu.sync_copy(rows_v, agg_out.at[c, pl.ds(r, _CHUNK)])
            pltpu.sync_copy(cnt_sh.at[pl.ds(r, _CHUNK)], rows_v)
            pltpu.sync_copy(rows_v, cnt_out.at[c, pl.ds(r, _CHUNK)])
        return carry

    lax.fori_loop(0, -(-_ZCH // _NS), dbody, 0)


def _scatter_call(msgs, dst, zeros128, ones128):
    mesh = plsc.VectorSubcoreMesh(core_axis_name="c", subcore_axis_name="s", num_cores=_NC, num_subcores=_NS)
    f = pl.kernel(
        _scatter_body,
        out_type=[
            jax.ShapeDtypeStruct((_NC, _TBL, HID), jnp.float32),
            jax.ShapeDtypeStruct((_NC, _TBL, HID), jnp.float32),
        ],
        mesh=mesh,
        scratch_types=[
            pltpu.VMEM((_CHUNK,), jnp.int32),
            pltpu.VMEM((_CHUNK,), jnp.int32),
            pltpu.VMEM((_CHUNK, HID), jnp.float32),
            pltpu.VMEM((_CHUNK, HID), jnp.float32),
            pltpu.VMEM_SHARED((_TBL, HID), jnp.float32),
            pltpu.VMEM_SHARED((_TBL, HID), jnp.float32),
        ],
    )
    return f(msgs, dst, zeros128, ones128)


def _final_body(x_ref, aggp_ref, cntp_ref,
                rcWa_ref, rcWb_ref, rcb_ref, g_ref, b_ref,
                ffW1_ref, ffb1_ref, ffW2_ref, ffb2_ref, o_ref):
    f32 = jnp.float32
    xv = x_ref[...]
    agg = aggp_ref[0]
    cnt = cntp_ref[0, :, 0:1]
    out = agg / jnp.maximum(cnt, 1.0)
    x1 = xv + out
    x2 = x1 + jnp.dot(x1, rcWa_ref[...], preferred_element_type=f32) \
            + jnp.dot(xv, rcWb_ref[...], preferred_element_type=f32) \
            + rcb_ref[...]
    xn2 = _ln_v(x2, g_ref[...], b_ref[...])
    hh = _gelu(jnp.dot(xn2, ffW1_ref[...], preferred_element_type=f32)
               + ffb1_ref[...])
    o_ref[...] = x2 + jnp.dot(hh, ffW2_ref[...], preferred_element_type=f32) \
                    + ffb2_ref[...]


def _final_call(x, aggp, cntp, rcWa, rcWb, rcb, g, b, ffW1, ffb1, ffW2, ffb2):
    grid = (N_NODES // _FB,)
    nblk = _NPS // _FB
    zero2 = lambda i: (0, 0)
    full = lambda a: pl.BlockSpec(a.shape, zero2)
    return pl.pallas_call(
        _final_body,
        grid=grid,
        in_specs=[
            pl.BlockSpec((_FB, HID), lambda i: (i, 0)),
            pl.BlockSpec((1, _FB, HID), lambda i: (i // nblk, i % nblk, 0)),
            pl.BlockSpec((1, _FB, HID), lambda i: (i // nblk, i % nblk, 0)),
            full(rcWa), full(rcWb), full(rcb), full(g), full(b),
            full(ffW1), full(ffb1), full(ffW2), full(ffb2),
        ],
        out_specs=pl.BlockSpec((_FB, HID), lambda i: (i, 0)),
        out_shape=jax.ShapeDtypeStruct((N_NODES, HID), jnp.float32),
        compiler_params=pltpu.CompilerParams(
            dimension_semantics=("arbitrary",)),
    )(x, aggp, cntp, rcWa, rcWb, rcb, g, b, ffW1, ffb1, ffW2, ffb2)


def kernel(x, edge_index, edge_attr, ln_g, ln_b,
           q_W1, q_b1, q_W2, q_b2,
           k_W1, k_b1, k_W2, k_b2,
           v_W1, v_b1, v_W2, v_b2,
           sp_W1, sp_b1, sp_W2, sp_b2,
           rc_W, rc_b,
           ff_W1, ff_b1, ff_W2, ff_b2):
    f32 = jnp.float32
    g2 = ln_g.reshape(1, HID)
    b2 = ln_b.reshape(1, HID)

    xn = _ln_call(x, g2, b2)

    src = edge_index[0]
    dst = edge_index[1]
    srcf, dstf = _gather_call(xn, src, dst)

    W1s = jnp.concatenate([q_W1[:HID], k_W1[:HID], v_W1[:HID]], axis=1)
    W1d = jnp.concatenate([q_W1[HID:2 * HID], k_W1[HID:2 * HID],
                           v_W1[HID:2 * HID]], axis=1)
    W1m = jnp.zeros((16, 3 * 2 * HID), f32).at[0:3].set(
        jnp.concatenate([q_W1[2 * HID:], k_W1[2 * HID:], v_W1[2 * HID:]],
                        axis=1))
    b1 = jnp.concatenate([q_b1, k_b1, v_b1]).reshape(1, 3 * 2 * HID)
    spW1 = jnp.zeros((16, 64), f32).at[3:7].set(sp_W1)
    spb1 = sp_b1.reshape(1, 64)
    spW2 = sp_W2
    spb2 = sp_b2.reshape(1, NH)

    i128 = lax.broadcasted_iota(jnp.int32, (HID, NH), 0)
    j8 = lax.broadcasted_iota(jnp.int32, (HID, NH), 1)
    R = (i128 // DPH == j8).astype(f32)
    i64 = lax.broadcasted_iota(jnp.int32, (NH * NH, NH), 0)
    j8b = lax.broadcasted_iota(jnp.int32, (NH * NH, NH), 1)
    D = (i64 // NH == j8b).astype(f32)
    Eexp = R.T

    msgs = _edge_call(srcf, dstf, edge_attr,
                      (W1s, W1d, W1m, b1,
                       q_W2, q_b2.reshape(1, HID),
                       k_W2, k_b2.reshape(1, HID),
                       v_W2, v_b2.reshape(1, HID),
                       spW1, spb1, spW2, spb2, R, D, Eexp))

    zeros128 = jnp.zeros((_CHUNK, HID), f32)
    ones128 = jnp.ones((_CHUNK, HID), f32)
    aggp, cntp = _scatter_call(msgs, dst, zeros128, ones128)

    return _final_call(x, aggp, cntp,
                       rc_W[:HID], rc_W[HID:], rc_b.reshape(1, HID),
                       g2, b2,
                       ff_W1, ff_b1.reshape(1, 2 * HID),
                       ff_W2, ff_b2.reshape(1, HID))
